# baseline (speedup 1.0000x reference)
import functools

import jax
import jax.numpy as jnp
import numpy as np
from jax import lax
from jax.experimental import pallas as pl
from jax.experimental.pallas import tpu as pltpu
from jax.experimental.pallas import tpu_sc as plsc

_NC = 2
_NS = 16
_NW = _NC * _NS
_LANES = 16
_CK = 128
_NB = 4


def _pe_const(length, dim):
  pos = np.arange(length, dtype=np.float32)[:, None]
  div = (1.0 / np.power(10000.0,
                        np.arange(0, dim * 2, 2, dtype=np.float32) / dim))
  pe = (pos * div[None, :]).astype(np.float32)
  pe[:, 0::2] = np.sin(pe[:, 0::2])
  pe[:, 1::2] = np.cos(pe[:, 1::2])
  return jnp.asarray(pe)


@functools.partial(jax.jit, static_argnames=("l_len", "batch", "dim"))
def _sc_lookup(idx, table, pe, *, l_len, batch, dim):
  rows_total = l_len * batch
  per_w = rows_total // _NW
  nch = per_w // _CK
  ch_per_pos = batch // _CK
  main = (nch // _NB) * _NB

  idx3 = idx.reshape(_NW, nch, _CK)
  mesh = plsc.VectorSubcoreMesh(core_axis_name="c", subcore_axis_name="s")

  @functools.partial(
      pl.kernel,
      out_type=jax.ShapeDtypeStruct((rows_total, dim), jnp.float32),
      mesh=mesh,
      scratch_types=(
          [pltpu.VMEM((nch, _CK), jnp.int32),
           pltpu.VMEM((l_len, dim), jnp.float32)]
          + [pltpu.VMEM((_CK, dim), jnp.float32) for _ in range(_NB)]
          + [pltpu.SemaphoreType.DMA for _ in range(2 * _NB)]
      ),
      compiler_params=pltpu.CompilerParams(use_tc_tiling_on_sc=False),
  )
  def run(idx_hbm, table_hbm, pe_hbm, out_hbm, idx_v, pe_v, *rest):
    bufs = rest[:_NB]
    gsems = rest[_NB:2 * _NB]
    osems = rest[2 * _NB:]
    wid = lax.axis_index("s") * _NC + lax.axis_index("c")
    base = wid * per_w

    pltpu.sync_copy(idx_hbm.at[wid], idx_v)
    pltpu.sync_copy(pe_hbm, pe_v)

    def gather_desc(cc, b):
      return pltpu.make_async_copy(table_hbm.at[idx_v.at[cc]], bufs[b],
                                   gsems[b])

    def out_desc(cc, b):
      return pltpu.make_async_copy(
          bufs[b], out_hbm.at[pl.ds(base + cc * _CK, _CK)], osems[b])

    def pe_add(cc, b):
      l = (wid * nch + cc) // ch_per_pos
      pes = [pe_v[l, pl.ds(k * _LANES, _LANES)] for k in range(dim // _LANES)]

      @plsc.parallel_loop(0, _CK, unroll=8)
      def _(r):
        for k in range(dim // _LANES):
          plsc.addupdate(bufs[b].at[r, pl.ds(k * _LANES, _LANES)], pes[k])

    def step(cc, b):
      gather_desc(cc, b).wait()
      pe_add(cc, b)
      out_desc(cc, b).start()

      @pl.when(cc >= 1)
      def _():
        out_desc(cc - 1, (b - 1) % _NB).wait()

      @pl.when(cc + (_NB - 1) < nch)
      def _():
        gather_desc(cc + (_NB - 1), (b + _NB - 1) % _NB).start()

    for b in range(_NB - 1):
      gather_desc(b, b).start()

    @pl.loop(0, main, step=_NB)
    def _(cc0):
      for b in range(_NB):
        step(cc0 + b, b)

    for cc in range(main, nch):
      b = cc % _NB
      gather_desc(cc, b).wait()
      pe_add(cc, b)
      out_desc(cc, b).start()
      out_desc(cc - 1, (b - 1) % _NB).wait()
      if cc + (_NB - 1) < nch:
        gather_desc(cc + (_NB - 1), (b + _NB - 1) % _NB).start()
    out_desc(nch - 1, (nch - 1) % _NB).wait()

  return run(idx3, table, pe)


def kernel(input, table):
  l_len, batch, _ = input.shape
  vocab, dim = table.shape
  idx = input.reshape(l_len * batch)
  pe = _pe_const(l_len, dim)
  out = _sc_lookup(idx, table, pe, l_len=l_len, batch=batch, dim=dim)
  return out.reshape(l_len, batch, dim)


# speedup vs baseline: 1.4299x; 1.4299x over previous
import functools

import jax
import jax.numpy as jnp
import numpy as np
from jax import lax
from jax.experimental import pallas as pl
from jax.experimental.pallas import tpu as pltpu
from jax.experimental.pallas import tpu_sc as plsc

_NC = 2
_NS = 16
_NW = _NC * _NS
_LANES = 16
_CK = 128
_NB = 4


def _pe_const(length, dim):
  pos = np.arange(length, dtype=np.float32)[:, None]
  div = (1.0 / np.power(10000.0,
                        np.arange(0, dim * 2, 2, dtype=np.float32) / dim))
  pe = (pos * div[None, :]).astype(np.float32)
  pe[:, 0::2] = np.sin(pe[:, 0::2])
  pe[:, 1::2] = np.cos(pe[:, 1::2])
  return jnp.asarray(pe.reshape(length // 2, 2 * dim))


@functools.partial(jax.jit, static_argnames=("l_len", "batch", "dim"))
def _sc_lookup(idx, table, pe, *, l_len, batch, dim):
  rows_total = l_len * batch
  per_w = rows_total // _NW
  nch = per_w // _CK
  ch_per_pos = batch // _CK
  main = (nch // _NB) * _NB

  mesh = plsc.VectorSubcoreMesh(core_axis_name="c", subcore_axis_name="s")

  @functools.partial(
      pl.kernel,
      out_type=jax.ShapeDtypeStruct((l_len, batch, dim), jnp.float32),
      mesh=mesh,
      scratch_types=(
          [pltpu.VMEM((l_len // 2, 2 * dim), jnp.float32)]
          + [pltpu.VMEM((_CK,), jnp.int32) for _ in range(_NB)]
          + [pltpu.VMEM((_CK, dim), jnp.float32) for _ in range(_NB)]
          + [pltpu.SemaphoreType.DMA for _ in range(3 * _NB)]
      ),
  )
  def run(idx_hbm, table_hbm, pe_hbm, out_hbm, pe_v, *rest):
    idxv = rest[:_NB]
    bufs = rest[_NB:2 * _NB]
    isems = rest[2 * _NB:3 * _NB]
    gsems = rest[3 * _NB:4 * _NB]
    osems = rest[4 * _NB:]
    wid = lax.axis_index("s") * _NC + lax.axis_index("c")

    pltpu.sync_copy(pe_hbm, pe_v)

    def chunk_pos(cc):
      g = wid * nch + cc
      return g // ch_per_pos, (g % ch_per_pos) * _CK

    def idx_desc(cc, b):
      l, b0 = chunk_pos(cc)
      return pltpu.make_async_copy(idx_hbm.at[l, pl.ds(b0, _CK)], idxv[b],
                                   isems[b])

    def rows_start(b):
      for g in range(_CK // _LANES):
        v = idxv[b][pl.ds(g * _LANES, _LANES)]
        for j in range(_LANES):
          pltpu.async_copy(table_hbm.at[v[j]], bufs[b].at[g * _LANES + j],
                           gsems[b])

    def rows_drain(b):
      pltpu.make_async_copy(table_hbm.at[pl.ds(0, _CK)], bufs[b],
                            gsems[b]).wait()

    def out_desc(cc, b):
      l, b0 = chunk_pos(cc)
      return pltpu.make_async_copy(bufs[b], out_hbm.at[l, pl.ds(b0, _CK)],
                                   osems[b])

    def pe_add(cc, b):
      l, _ = chunk_pos(cc)
      half = (l % 2) * dim
      pes = [pe_v[l // 2, pl.ds(half + k * _LANES, _LANES)]
             for k in range(dim // _LANES)]

      @plsc.parallel_loop(0, _CK, unroll=8)
      def _(r):
        for k in range(dim // _LANES):
          plsc.addupdate(bufs[b].at[r, pl.ds(k * _LANES, _LANES)], pes[k])

    def step(cc, b, tail):
      idx_desc(cc, b).wait()
      rows_start(b)
      rows_drain(b)
      pe_add(cc, b)
      out_desc(cc, b).start()
      if tail:
        out_desc(cc - 1, (b - 1) % _NB).wait()
        if cc + (_NB - 1) < nch:
          idx_desc(cc + (_NB - 1), (b + _NB - 1) % _NB).start()
      else:
        @pl.when(cc >= 1)
        def _():
          out_desc(cc - 1, (b - 1) % _NB).wait()

        @pl.when(cc + (_NB - 1) < nch)
        def _():
          idx_desc(cc + (_NB - 1), (b + _NB - 1) % _NB).start()

    for b in range(_NB - 1):
      idx_desc(b, b).start()

    @pl.loop(0, main, step=_NB)
    def _(cc0):
      for b in range(_NB):
        step(cc0 + b, b, False)

    for cc in range(main, nch):
      step(cc, cc % _NB, True)
    out_desc(nch - 1, (nch - 1) % _NB).wait()

  return run(idx, table, pe)


def kernel(input, table):
  l_len, batch, _ = input.shape
  vocab, dim = table.shape
  idx = input[:, :, 0]
  pe = _pe_const(l_len, dim)
  return _sc_lookup(idx, table, pe, l_len=l_len, batch=batch, dim=dim)
